# Initial kernel scaffold; baseline (speedup 1.0000x reference)
import jax
import jax.numpy as jnp
from jax.experimental import pallas as pl


def kernel(u, i, edge_index, user_emb, entity_emb, W1_self, W1_neigh, b1, W2_self, W2_neigh, b2):
    raise NotImplementedError("write your pallas kernel here")


# speedup vs baseline: 3.0493x; 3.0493x over previous
import functools

import jax
import jax.numpy as jnp
from jax import lax
from jax.experimental import pallas as pl
from jax.experimental.pallas import tpu as pltpu
from jax.experimental.pallas import tpu_sc as plsc

N = 10000
D = 128
DH = D // 2
NPAD = 10240
E = 320000
EPAD = 327680
B = 4096

NC = 2
NS = 16
NW = NC * NS
CH = 128
EPT = EPAD // NS
NCHUNK = EPT // CH
RPT = NPAD // NS
BPW = B // NW

_mesh = plsc.VectorSubcoreMesh(core_axis_name="c", subcore_axis_name="s")
_f32 = jnp.float32


def _renorm(t):
    n = jnp.sqrt(jnp.sum(t * t, axis=-1, keepdims=True))
    scale = jnp.minimum(1.0, 1.0 / jnp.maximum(n, 1e-12))
    return t * scale


def _renorm_body(x_ref, lo_ref, hi_ref):
    x = _renorm(x_ref[...])
    lo_ref[...] = x[:, :DH]
    hi_ref[...] = x[:, DH:]


def _dense_layer(lo_ref, hi_ref, agg_ref, deg_ref, ws_ref, wn_ref, b_ref):
    x = jnp.concatenate([lo_ref[...], hi_ref[...]], axis=1)
    agg = jnp.concatenate([agg_ref[0], agg_ref[1]], axis=1)
    deg = deg_ref[:, 0:1]
    mean = agg / jnp.maximum(deg, 1.0)
    return (jnp.dot(x, ws_ref[...], preferred_element_type=_f32)
            + jnp.dot(mean, wn_ref[...], preferred_element_type=_f32)
            + b_ref[...])


def _layer1_body(lo_ref, hi_ref, agg_ref, deg_ref, ws_ref, wn_ref, b_ref,
                 olo_ref, ohi_ref):
    h = _dense_layer(lo_ref, hi_ref, agg_ref, deg_ref, ws_ref, wn_ref, b_ref)
    h = jnp.maximum(h, 0.0)
    rows = lax.broadcasted_iota(jnp.int32, (NPAD, 1), 0)
    h = jnp.where(rows < N, h, 0.0)
    olo_ref[...] = h[:, :DH]
    ohi_ref[...] = h[:, DH:]


def _layer2_body(lo_ref, hi_ref, agg_ref, deg_ref, ws_ref, wn_ref, b_ref,
                 uraw_ref, o_ref, un_ref):
    o_ref[...] = _dense_layer(lo_ref, hi_ref, agg_ref, deg_ref,
                              ws_ref, wn_ref, b_ref)
    un_ref[...] = _renorm(uraw_ref[...])


def _zero_vmem(ref, nrows, ncols):
    zero16 = jnp.zeros((16,), _f32)

    def zfill(r, carry):
        for j in range(ncols // 16):
            ref[r, pl.ds(j * 16, 16)] = zero16
        return carry

    lax.fori_loop(0, nrows, zfill, None)


def _edge_pass(x_half, src_hbm, dst_hbm, sidx, didx, rows, acc, gsem,
               s, with_deg, ones16, dacc):
    def chunk(t, carry):
        base = s * EPT + t * CH
        pltpu.sync_copy(src_hbm.at[pl.ds(base, CH)], sidx)
        pltpu.sync_copy(dst_hbm.at[pl.ds(base, CH)], didx)
        pltpu.async_copy(x_half.at[sidx], rows, gsem).wait()
        pltpu.sync_copy(rows, acc.at[didx], add=True)
        if with_deg:
            pltpu.sync_copy(ones16, dacc.at[didx], add=True)
        return carry

    lax.fori_loop(0, NCHUNK, chunk, None)


def _agg1_body(xlo_hbm, xhi_hbm, src_hbm, dst_hbm, u_hbm, uemb_hbm,
               agg_hbm, deg_hbm, uraw_hbm,
               sidx, didx, rows, ones16, zbuf, urows, acc, dacc, gsem):
    c = lax.axis_index("c")
    s = lax.axis_index("s")
    wid = s * NC + c

    _zero_vmem(zbuf, CH, DH)
    _zero_vmem(ones16, CH, 16)

    def zcopy(k, carry):
        base = s * RPT + k * CH
        pltpu.sync_copy(zbuf, acc.at[pl.ds(base, CH)])
        pltpu.sync_copy(ones16, dacc.at[pl.ds(base, CH)])
        return carry

    lax.fori_loop(0, RPT // CH, zcopy, None)

    one16 = jnp.ones((16,), _f32)

    def ofill(r, carry):
        ones16[r, pl.ds(0, 16)] = one16
        return carry

    lax.fori_loop(0, CH, ofill, None)
    plsc.subcore_barrier()

    @pl.when(c == 0)
    def _():
        _edge_pass(xlo_hbm, src_hbm, dst_hbm, sidx, didx, rows, acc, gsem,
                   s, True, ones16, dacc)

    @pl.when(c == 1)
    def _():
        _edge_pass(xhi_hbm, src_hbm, dst_hbm, sidx, didx, rows, acc, gsem,
                   s, False, ones16, dacc)

    ub = wid * BPW
    pltpu.sync_copy(u_hbm.at[pl.ds(ub, BPW)], sidx)
    pltpu.async_copy(uemb_hbm.at[sidx], urows, gsem).wait()
    pltpu.sync_copy(urows, uraw_hbm.at[pl.ds(ub, BPW)])

    plsc.subcore_barrier()
    ob = s * RPT
    pltpu.sync_copy(acc.at[pl.ds(ob, RPT)], agg_hbm.at[c, pl.ds(ob, RPT)])

    @pl.when(c == 0)
    def _():
        pltpu.sync_copy(dacc.at[pl.ds(ob, RPT)], deg_hbm.at[pl.ds(ob, RPT)])


def _agg2_body(hlo_hbm, hhi_hbm, src_hbm, dst_hbm, agg_hbm,
               sidx, didx, rows, zbuf, acc, gsem):
    c = lax.axis_index("c")
    s = lax.axis_index("s")

    _zero_vmem(zbuf, CH, DH)

    def zcopy(k, carry):
        pltpu.sync_copy(zbuf, acc.at[pl.ds(s * RPT + k * CH, CH)])
        return carry

    lax.fori_loop(0, RPT // CH, zcopy, None)
    plsc.subcore_barrier()

    @pl.when(c == 0)
    def _():
        _edge_pass(hlo_hbm, src_hbm, dst_hbm, sidx, didx, rows, acc, gsem,
                   s, False, None, None)

    @pl.when(c == 1)
    def _():
        _edge_pass(hhi_hbm, src_hbm, dst_hbm, sidx, didx, rows, acc, gsem,
                   s, False, None, None)

    plsc.subcore_barrier()
    ob = s * RPT
    pltpu.sync_copy(acc.at[pl.ds(ob, RPT)], agg_hbm.at[c, pl.ds(ob, RPT)])


def _final_body(o2_hbm, un_hbm, i_hbm, o_hbm,
                iidx, items, urows, res, gsem):
    c = lax.axis_index("c")
    s = lax.axis_index("s")
    wid = s * NC + c
    base = wid * BPW

    pltpu.sync_copy(i_hbm.at[pl.ds(base, BPW)], iidx)
    pltpu.async_copy(o2_hbm.at[iidx], items, gsem).wait()
    pltpu.sync_copy(un_hbm.at[pl.ds(base, BPW)], urows)

    lane = lax.iota(jnp.int32, 16)

    def dotgroup(g, carry):
        out16 = jnp.zeros((16,), _f32)
        for r16 in range(16):
            r = g * 16 + r16
            accv = jnp.zeros((16,), _f32)
            for j in range(D // 16):
                accv = accv + (items[r, pl.ds(j * 16, 16)]
                               * urows[r, pl.ds(j * 16, 16)])
            out16 = jnp.where(lane == r16, jnp.sum(accv), out16)
        res[pl.ds(g * 16, 16)] = 1.0 / (1.0 + jnp.exp(-out16))
        return carry

    lax.fori_loop(0, BPW // 16, dotgroup, None)
    pltpu.sync_copy(res, o_hbm.at[pl.ds(base, BPW)])


_sc_params = pltpu.CompilerParams(use_tc_tiling_on_sc=False, needs_layout_passes=False)

_sc_pass1 = pl.kernel(
    _agg1_body,
    compiler_params=_sc_params,
    out_type=(
        jax.ShapeDtypeStruct((NC, NPAD, DH), _f32),
        jax.ShapeDtypeStruct((NPAD, 16), _f32),
        jax.ShapeDtypeStruct((B, D), _f32),
    ),
    mesh=_mesh,
    scratch_types=[
        pltpu.VMEM((CH,), jnp.int32),
        pltpu.VMEM((CH,), jnp.int32),
        pltpu.VMEM((CH, DH), _f32),
        pltpu.VMEM((CH, 16), _f32),
        pltpu.VMEM((CH, DH), _f32),
        pltpu.VMEM((BPW, D), _f32),
        pltpu.VMEM_SHARED((NPAD, DH), _f32),
        pltpu.VMEM_SHARED((NPAD, 16), _f32),
        pltpu.SemaphoreType.DMA,
    ],
)

_sc_pass2 = pl.kernel(
    _agg2_body,
    compiler_params=_sc_params,
    out_type=jax.ShapeDtypeStruct((NC, NPAD, DH), _f32),
    mesh=_mesh,
    scratch_types=[
        pltpu.VMEM((CH,), jnp.int32),
        pltpu.VMEM((CH,), jnp.int32),
        pltpu.VMEM((CH, DH), _f32),
        pltpu.VMEM((CH, DH), _f32),
        pltpu.VMEM_SHARED((NPAD, DH), _f32),
        pltpu.SemaphoreType.DMA,
    ],
)

_sc_final = pl.kernel(
    _final_body,
    compiler_params=_sc_params,
    out_type=jax.ShapeDtypeStruct((B,), _f32),
    mesh=_mesh,
    scratch_types=[
        pltpu.VMEM((BPW,), jnp.int32),
        pltpu.VMEM((BPW, D), _f32),
        pltpu.VMEM((BPW, D), _f32),
        pltpu.VMEM((BPW,), _f32),
        pltpu.SemaphoreType.DMA,
    ],
)

_tc_renorm = pl.pallas_call(
    _renorm_body,
    out_shape=(
        jax.ShapeDtypeStruct((NPAD, DH), _f32),
        jax.ShapeDtypeStruct((NPAD, DH), _f32),
    ),
)

_tc_layer1 = pl.pallas_call(
    _layer1_body,
    out_shape=(
        jax.ShapeDtypeStruct((NPAD, DH), _f32),
        jax.ShapeDtypeStruct((NPAD, DH), _f32),
    ),
)

_tc_layer2 = pl.pallas_call(
    _layer2_body,
    out_shape=(
        jax.ShapeDtypeStruct((NPAD, D), _f32),
        jax.ShapeDtypeStruct((B, D), _f32),
    ),
)


def kernel(u, i, edge_index, user_emb, entity_emb,
           W1_self, W1_neigh, b1, W2_self, W2_neigh, b2):
    src = edge_index[0].astype(jnp.int32)
    dst = edge_index[1].astype(jnp.int32)
    pad = jnp.full((EPAD - E,), N, jnp.int32)
    srcp = jnp.concatenate([src, pad])
    dstp = jnp.concatenate([dst, pad])
    u32 = u.astype(jnp.int32)
    i32 = i.astype(jnp.int32)

    x_in = jnp.pad(entity_emb.astype(_f32), ((0, NPAD - N), (0, 0)))
    xlo, xhi = _tc_renorm(x_in)

    agg1, deg, uraw = _sc_pass1(xlo, xhi, srcp, dstp, u32,
                                user_emb.astype(_f32))
    hlo, hhi = _tc_layer1(xlo, xhi, agg1, deg, W1_self, W1_neigh,
                          b1.reshape(1, D))
    agg2 = _sc_pass2(hlo, hhi, srcp, dstp)
    out2, un = _tc_layer2(hlo, hhi, agg2, deg, W2_self, W2_neigh,
                          b2.reshape(1, D), uraw)
    return _sc_final(out2, un, i32)
